# Initial kernel scaffold; baseline (speedup 1.0000x reference)
import jax
import jax.numpy as jnp
from jax.experimental import pallas as pl


def kernel(users, items, edge_index, user_emb, item_emb, Wmu, bmu, Wlv, blv, Wdec, bdec, attW, attb, eps):
    raise NotImplementedError("write your pallas kernel here")


# speedup vs baseline: 7.6615x; 7.6615x over previous
import functools

import jax
import jax.numpy as jnp
from jax import lax
from jax.experimental import pallas as pl
from jax.experimental.pallas import tpu as pltpu
from jax.experimental.pallas import tpu_sc as plsc

_USER = 20000
_ITEM = 30000
_N = 50000
_E = 800000
_D = 64
_L = 3
_B = 4096

_NC = 2
_NS = 16

_CPE = 128
_ROWS = _E // _CPE
_ROWS_PAD = 6400
_TRASH = _N
_NCNT = 51200
_NACC = 50048
_NPT = _NACC // _NS

_DEG_RPT = _ROWS_PAD // (_NC * _NS)
_SP_RPT = _ROWS_PAD // _NS
_SP_STG = 80
_SP_NSTG = _SP_RPT // _SP_STG

_mesh = plsc.VectorSubcoreMesh(
    core_axis_name="c", subcore_axis_name="s", num_cores=_NC, num_subcores=_NS
)


def _deg_body(row2d, out, stage_v, ones_v, zero_v, cnt_sh):
    c = lax.axis_index("c")
    s = lax.axis_index("s")
    w = s * _NC + c

    @pl.loop(0, 200)
    def _fill_zero(i):
        zero_v[pl.ds(i * 16, 16)] = jnp.zeros((16,), jnp.float32)

    @pl.loop(0, 8)
    def _fill_one(i):
        ones_v[pl.ds(i * 16, 16)] = jnp.ones((16,), jnp.float32)

    pltpu.sync_copy(zero_v, cnt_sh.at[pl.ds(s * 3200, 3200)])
    plsc.subcore_barrier()

    pltpu.sync_copy(row2d.at[pl.ds(w * _DEG_RPT, _DEG_RPT)], stage_v)

    @pl.loop(0, _DEG_RPT)
    def _count(k):
        pltpu.sync_copy(ones_v, cnt_sh.at[stage_v.at[k]], add=True)

    plsc.subcore_barrier()
    pltpu.sync_copy(cnt_sh.at[pl.ds(s * 3200, 3200)], out.at[c, pl.ds(s * 3200, 3200)])


_deg_call = pl.kernel(
    _deg_body,
    out_type=jax.ShapeDtypeStruct((_NC, _NCNT), jnp.float32),
    mesh=_mesh,
    compiler_params=pltpu.CompilerParams(use_tc_tiling_on_sc=False),
    scratch_types=[
        pltpu.VMEM((_DEG_RPT, _CPE), jnp.int32),
        pltpu.VMEM((_CPE,), jnp.float32),
        pltpu.VMEM((3200,), jnp.float32),
        pltpu.VMEM_SHARED((_NCNT,), jnp.float32),
    ],
)


def _spmm_body(xs, col2d, row2d, out, colstage, rowstage, rows_v, zrow_v, acc_sh, sem):
    c = lax.axis_index("c")
    s = lax.axis_index("s")
    base = s * _SP_RPT

    @pl.loop(0, 136)
    def _fill_zero(i):
        zrow_v[i, pl.ds(0, 16)] = jnp.zeros((16,), jnp.float32)
        zrow_v[i, pl.ds(16, 16)] = jnp.zeros((16,), jnp.float32)

    @pl.loop(0, 23)
    def _zero_acc(j):
        pltpu.sync_copy(zrow_v, acc_sh.at[pl.ds(s * _NPT + j * 136, 136)])

    plsc.subcore_barrier()

    for t in range(_SP_NSTG):
        pltpu.sync_copy(col2d.at[pl.ds(base + t * _SP_STG, _SP_STG)], colstage)
        pltpu.sync_copy(row2d.at[pl.ds(base + t * _SP_STG, _SP_STG)], rowstage)

        @pl.loop(0, _SP_STG)
        def _edges(k):
            pltpu.async_copy(xs.at[c].at[colstage.at[k]], rows_v, sem).wait()
            pltpu.sync_copy(rows_v, acc_sh.at[rowstage.at[k]], add=True)

    plsc.subcore_barrier()
    pltpu.sync_copy(
        acc_sh.at[pl.ds(s * _NPT, _NPT)], out.at[c, pl.ds(s * _NPT, _NPT)]
    )


_spmm_call = pl.kernel(
    _spmm_body,
    out_type=jax.ShapeDtypeStruct((_NC, _NACC, 32), jnp.float32),
    mesh=_mesh,
    compiler_params=pltpu.CompilerParams(use_tc_tiling_on_sc=False),
    scratch_types=[
        pltpu.VMEM((_SP_STG, _CPE), jnp.int32),
        pltpu.VMEM((_SP_STG, _CPE), jnp.int32),
        pltpu.VMEM((_CPE, 32), jnp.float32),
        pltpu.VMEM((136, 32), jnp.float32),
        pltpu.VMEM_SHARED((_NACC, 32), jnp.float32),
        pltpu.SemaphoreType.DMA,
    ],
)


def _gather_body(final, uidx, vidx, out, iu_v, iv_v, u_v, v_v, sem):
    c = lax.axis_index("c")
    s = lax.axis_index("s")
    w = s * _NC + c
    npair = _B // (_NC * _NS)
    base = w * npair

    pltpu.sync_copy(uidx.at[pl.ds(base, npair)], iu_v)
    pltpu.sync_copy(vidx.at[pl.ds(base, npair)], iv_v)
    cp_u = pltpu.async_copy(final.at[iu_v], u_v, sem)
    cp_v = pltpu.async_copy(final.at[iv_v], v_v, sem)
    cp_u.wait()
    cp_v.wait()
    pltpu.sync_copy(u_v, out.at[0, pl.ds(base, npair)])
    pltpu.sync_copy(v_v, out.at[1, pl.ds(base, npair)])


_gather_call = pl.kernel(
    _gather_body,
    out_type=jax.ShapeDtypeStruct((2, _B, _D), jnp.float32),
    mesh=_mesh,
    compiler_params=pltpu.CompilerParams(use_tc_tiling_on_sc=False),
    scratch_types=[
        pltpu.VMEM((_B // (_NC * _NS),), jnp.int32),
        pltpu.VMEM((_B // (_NC * _NS),), jnp.int32),
        pltpu.VMEM((_B // (_NC * _NS), _D), jnp.float32),
        pltpu.VMEM((_B // (_NC * _NS), _D), jnp.float32),
        pltpu.SemaphoreType.DMA,
    ],
)


def _vae_terms(x, Wmu_i, bmu_i, Wlv_i, blv_i, Wdec_i, bdec_i, eps_i):
    mu = x @ Wmu_i + bmu_i
    logvar = x @ Wlv_i + blv_i
    std = jnp.exp(0.5 * logvar)
    z = mu + eps_i * std
    recon = z @ Wdec_i + bdec_i
    recon_loss = jnp.mean((recon - x) ** 2)
    kl_loss = -0.5 * jnp.mean(1.0 + logvar - mu**2 - jnp.exp(logvar))
    return recon_loss + kl_loss


def kernel(users, items, edge_index, user_emb, item_emb, Wmu, bmu, Wlv, blv, Wdec, bdec, attW, attb, eps):
    row = edge_index[0]
    col = edge_index[1]
    row2d = jnp.pad(
        row.reshape(_ROWS, _CPE),
        ((0, _ROWS_PAD - _ROWS), (0, 0)),
        constant_values=_TRASH,
    )
    col2d = jnp.pad(
        col.reshape(_ROWS, _CPE), ((0, _ROWS_PAD - _ROWS), (0, 0)), constant_values=0
    )

    cnt2 = _deg_call(row2d)
    cnt = cnt2[0, :_N] + cnt2[1, :_N]
    dinv = jnp.power(cnt + 1e-10, -0.5)

    x0 = jnp.concatenate([user_emb, item_emb], axis=0)
    xsn = dinv[:, None] * x0
    xs = jnp.stack([xsn[:, :32], xsn[:, 32:]])

    embs = [x0]
    x = x0
    for _ in range(_L):
        acc2 = _spmm_call(xs, col2d, row2d)
        acc = jnp.concatenate([acc2[0, :_N], acc2[1, :_N]], axis=1)
        x = dinv[:, None] * acc
        embs.append(x)
        xsn = dinv[:, None] * x
        xs = jnp.stack([xsn[:, :32], xsn[:, 32:]])

    vae_losses = [
        _vae_terms(embs[i + 1], Wmu[i], bmu[i], Wlv[i], blv[i], Wdec[i], bdec[i], eps[i])
        for i in range(_L)
    ]
    vae_losses.append(
        _vae_terms(embs[_L], Wmu[_L], bmu[_L], Wlv[_L], blv[_L], Wdec[_L], bdec[_L], eps[_L])
    )

    embs_tensor = jnp.stack(embs, axis=1)
    embs_flat = embs_tensor.reshape(_N, -1)
    logits = embs_flat @ attW + attb
    scores_att = jax.nn.softmax(logits, axis=1)[:, :, None]
    final = jnp.sum(embs_tensor * scores_att, axis=1)

    vidx = items + _USER
    uv = _gather_call(final, users, vidx)
    scores = jnp.sum(uv[0] * uv[1], axis=1)
    return scores, jnp.mean(jnp.stack(vae_losses))


# speedup vs baseline: 8.7672x; 1.1443x over previous
import functools

import jax
import jax.numpy as jnp
from jax import lax
from jax.experimental import pallas as pl
from jax.experimental.pallas import tpu as pltpu
from jax.experimental.pallas import tpu_sc as plsc

_USER = 20000
_ITEM = 30000
_N = 50000
_E = 800000
_D = 64
_L = 3
_B = 4096

_NC = 2
_NS = 16

_CPE = 128
_ROWS = _E // _CPE
_ROWS_PAD = 6400
_TRASH = _N
_NCNT = 51200
_NACC = 50048
_NPT = _NACC // _NS

_DEG_RPT = _ROWS_PAD // (_NC * _NS)
_SP_RPT = _ROWS_PAD // _NS
_SP_STG = 80
_SP_NSTG = _SP_RPT // _SP_STG

_mesh = plsc.VectorSubcoreMesh(
    core_axis_name="c", subcore_axis_name="s", num_cores=_NC, num_subcores=_NS
)


def _deg_body(row2d, out, stage_v, ones_v, zero_v, cnt_sh):
    c = lax.axis_index("c")
    s = lax.axis_index("s")
    w = s * _NC + c

    @pl.loop(0, 200)
    def _fill_zero(i):
        zero_v[pl.ds(i * 16, 16)] = jnp.zeros((16,), jnp.float32)

    @pl.loop(0, 8)
    def _fill_one(i):
        ones_v[pl.ds(i * 16, 16)] = jnp.ones((16,), jnp.float32)

    pltpu.sync_copy(zero_v, cnt_sh.at[pl.ds(s * 3200, 3200)])
    plsc.subcore_barrier()

    pltpu.sync_copy(row2d.at[pl.ds(w * _DEG_RPT, _DEG_RPT)], stage_v)

    @pl.loop(0, _DEG_RPT)
    def _count(k):
        pltpu.sync_copy(ones_v, cnt_sh.at[stage_v.at[k]], add=True)

    plsc.subcore_barrier()
    pltpu.sync_copy(cnt_sh.at[pl.ds(s * 3200, 3200)], out.at[c, pl.ds(s * 3200, 3200)])


_deg_call = pl.kernel(
    _deg_body,
    out_type=jax.ShapeDtypeStruct((_NC, _NCNT), jnp.float32),
    mesh=_mesh,
    compiler_params=pltpu.CompilerParams(use_tc_tiling_on_sc=False, needs_layout_passes=False),
    scratch_types=[
        pltpu.VMEM((_DEG_RPT, _CPE), jnp.int32),
        pltpu.VMEM((_CPE,), jnp.float32),
        pltpu.VMEM((3200,), jnp.float32),
        pltpu.VMEM_SHARED((_NCNT,), jnp.float32),
    ],
)


def _spmm_body(xs, col2d, row2d, out, colstage, rowstage, rows_v, zrow_v, acc_sh, sem):
    c = lax.axis_index("c")
    s = lax.axis_index("s")
    base = s * _SP_RPT

    @pl.loop(0, 136)
    def _fill_zero(i):
        zrow_v[i, pl.ds(0, 16)] = jnp.zeros((16,), jnp.float32)
        zrow_v[i, pl.ds(16, 16)] = jnp.zeros((16,), jnp.float32)

    @pl.loop(0, 23)
    def _zero_acc(j):
        pltpu.sync_copy(zrow_v, acc_sh.at[pl.ds(s * _NPT + j * 136, 136)])

    plsc.subcore_barrier()

    for t in range(_SP_NSTG):
        pltpu.sync_copy(col2d.at[pl.ds(base + t * _SP_STG, _SP_STG)], colstage)
        pltpu.sync_copy(row2d.at[pl.ds(base + t * _SP_STG, _SP_STG)], rowstage)

        @pl.loop(0, _SP_STG)
        def _edges(k):
            pltpu.async_copy(xs.at[c].at[colstage.at[k]], rows_v, sem).wait()
            pltpu.sync_copy(rows_v, acc_sh.at[rowstage.at[k]], add=True)

    plsc.subcore_barrier()
    pltpu.sync_copy(
        acc_sh.at[pl.ds(s * _NPT, _NPT)], out.at[c, pl.ds(s * _NPT, _NPT)]
    )


_spmm_call = pl.kernel(
    _spmm_body,
    out_type=jax.ShapeDtypeStruct((_NC, _NACC, 32), jnp.float32),
    mesh=_mesh,
    compiler_params=pltpu.CompilerParams(use_tc_tiling_on_sc=False, needs_layout_passes=False),
    scratch_types=[
        pltpu.VMEM((_SP_STG, _CPE), jnp.int32),
        pltpu.VMEM((_SP_STG, _CPE), jnp.int32),
        pltpu.VMEM((_CPE, 32), jnp.float32),
        pltpu.VMEM((136, 32), jnp.float32),
        pltpu.VMEM_SHARED((_NACC, 32), jnp.float32),
        pltpu.SemaphoreType.DMA,
    ],
)


def _score_body(final, uidx, vidx, out, iu_v, iv_v, u_v, v_v, sc_v, sem):
    c = lax.axis_index("c")
    s = lax.axis_index("s")
    w = s * _NC + c
    npair = _B // (_NC * _NS)
    base = w * npair

    pltpu.sync_copy(uidx.at[pl.ds(base, npair)], iu_v)
    pltpu.sync_copy(vidx.at[pl.ds(base, npair)], iv_v)
    cp_u = pltpu.async_copy(final.at[iu_v], u_v, sem)
    cp_v = pltpu.async_copy(final.at[iv_v], v_v, sem)
    cp_u.wait()
    cp_v.wait()

    lane = lax.iota(jnp.int32, 16)

    @pl.loop(0, npair // 16)
    def _grp(g):
        vec = jnp.zeros((16,), jnp.float32)
        for j in range(16):
            p = g * 16 + j
            d = jnp.sum(
                u_v[p, pl.ds(0, 16)] * v_v[p, pl.ds(0, 16)]
                + u_v[p, pl.ds(16, 16)] * v_v[p, pl.ds(16, 16)]
                + u_v[p, pl.ds(32, 16)] * v_v[p, pl.ds(32, 16)]
                + u_v[p, pl.ds(48, 16)] * v_v[p, pl.ds(48, 16)],
                axis=0,
            )
            vec = jnp.where(lane == j, d, vec)
        sc_v[pl.ds(g * 16, 16)] = vec

    pltpu.sync_copy(sc_v, out.at[pl.ds(base, npair)])


_score_call = pl.kernel(
    _score_body,
    out_type=jax.ShapeDtypeStruct((_B,), jnp.float32),
    mesh=_mesh,
    compiler_params=pltpu.CompilerParams(use_tc_tiling_on_sc=False, needs_layout_passes=False),
    scratch_types=[
        pltpu.VMEM((_B // (_NC * _NS),), jnp.int32),
        pltpu.VMEM((_B // (_NC * _NS),), jnp.int32),
        pltpu.VMEM((_B // (_NC * _NS), _D), jnp.float32),
        pltpu.VMEM((_B // (_NC * _NS), _D), jnp.float32),
        pltpu.VMEM((_B // (_NC * _NS),), jnp.float32),
        pltpu.SemaphoreType.DMA,
    ],
)

_R = 2000
_G = _N // _R


def _vae_sums(x, eps, wmu, bmu, wlv, blv, wdec, bdec):
    mu = x @ wmu + bmu
    lv = x @ wlv + blv
    std = jnp.exp(0.5 * lv)
    z = mu + eps * std
    recon = z @ wdec + bdec
    rsum = jnp.sum((recon - x) ** 2)
    ksum = jnp.sum(1.0 + lv - mu * mu - jnp.exp(lv))
    return rsum, ksum


def _sums_row(rsum, ksum):
    lanes = lax.broadcasted_iota(jnp.int32, (1, 1, 128), 2)
    return jnp.where(lanes == 0, rsum, jnp.where(lanes == 1, ksum, 0.0))


def _post_body(acc_ref, dinv_ref, eps_ref, wmu_ref, bmu_ref, wlv_ref, blv_ref,
               wdec_ref, bdec_ref, x_ref, xs_ref, sums_ref):
    acc = jnp.concatenate([acc_ref[0], acc_ref[1]], axis=1)
    dinv = dinv_ref[...]
    x = dinv * acc
    x_ref[...] = x
    xsn = dinv * x
    xs_ref[0] = xsn[:, :32]
    xs_ref[1] = xsn[:, 32:]
    rsum, ksum = _vae_sums(
        x, eps_ref[...], wmu_ref[...], bmu_ref[...], wlv_ref[...],
        blv_ref[...], wdec_ref[...], bdec_ref[...]
    )
    sums_ref[...] = _sums_row(rsum, ksum)


_wspec = pl.BlockSpec((_D, _D), lambda i: (0, 0))
_bspec = pl.BlockSpec((1, _D), lambda i: (0, 0))

_post_call = pl.pallas_call(
    _post_body,
    grid=(_G,),
    in_specs=[
        pl.BlockSpec((2, _R, 32), lambda i: (0, i, 0)),
        pl.BlockSpec((_R, 1), lambda i: (i, 0)),
        pl.BlockSpec((_R, _D), lambda i: (i, 0)),
        _wspec, _bspec, _wspec, _bspec, _wspec, _bspec,
    ],
    out_specs=[
        pl.BlockSpec((_R, _D), lambda i: (i, 0)),
        pl.BlockSpec((2, _R, 32), lambda i: (0, i, 0)),
        pl.BlockSpec((1, 1, 128), lambda i: (i, 0, 0)),
    ],
    out_shape=[
        jax.ShapeDtypeStruct((_N, _D), jnp.float32),
        jax.ShapeDtypeStruct((2, _N, 32), jnp.float32),
        jax.ShapeDtypeStruct((_G, 1, 128), jnp.float32),
    ],
)


def _final_body(x0_ref, x1_ref, x2_ref, x3_ref, eps_ref, wmu_ref, bmu_ref,
                wlv_ref, blv_ref, wdec_ref, bdec_ref, attw_ref, attbp_ref,
                final_ref, sums_ref):
    x0, x1, x2, x3 = x0_ref[...], x1_ref[...], x2_ref[...], x3_ref[...]
    flat = jnp.concatenate([x0, x1, x2, x3], axis=1)
    logits = flat @ attw_ref[...] + attbp_ref[...]
    m = jnp.max(logits, axis=1, keepdims=True)
    e = jnp.exp(logits - m)
    a = e / jnp.sum(e, axis=1, keepdims=True)
    final_ref[...] = (
        a[:, 0:1] * x0 + a[:, 1:2] * x1 + a[:, 2:3] * x2 + a[:, 3:4] * x3
    )
    rsum, ksum = _vae_sums(
        x3, eps_ref[...], wmu_ref[...], bmu_ref[...], wlv_ref[...],
        blv_ref[...], wdec_ref[...], bdec_ref[...]
    )
    sums_ref[...] = _sums_row(rsum, ksum)


_xspec = pl.BlockSpec((_R, _D), lambda i: (i, 0))

_final_call = pl.pallas_call(
    _final_body,
    grid=(_G,),
    in_specs=[
        _xspec, _xspec, _xspec, _xspec, _xspec,
        _wspec, _bspec, _wspec, _bspec, _wspec, _bspec,
        pl.BlockSpec(((_L + 1) * _D, 128), lambda i: (0, 0)),
        pl.BlockSpec((1, 128), lambda i: (0, 0)),
    ],
    out_specs=[
        pl.BlockSpec((_R, _D), lambda i: (i, 0)),
        pl.BlockSpec((1, 1, 128), lambda i: (i, 0, 0)),
    ],
    out_shape=[
        jax.ShapeDtypeStruct((_N, _D), jnp.float32),
        jax.ShapeDtypeStruct((_G, 1, 128), jnp.float32),
    ],
)


def kernel(users, items, edge_index, user_emb, item_emb, Wmu, bmu, Wlv, blv, Wdec, bdec, attW, attb, eps):
    row = edge_index[0]
    col = edge_index[1]
    row2d = jnp.pad(
        row.reshape(_ROWS, _CPE),
        ((0, _ROWS_PAD - _ROWS), (0, 0)),
        constant_values=_TRASH,
    )
    col2d = jnp.pad(
        col.reshape(_ROWS, _CPE), ((0, _ROWS_PAD - _ROWS), (0, 0)), constant_values=0
    )

    cnt2 = _deg_call(row2d)
    cnt = cnt2[0, :_N] + cnt2[1, :_N]
    dinv = jnp.power(cnt + 1e-10, -0.5)
    dinv2d = dinv[:, None]

    x0 = jnp.concatenate([user_emb, item_emb], axis=0)
    xsn = dinv2d * x0
    xs = jnp.stack([xsn[:, :32], xsn[:, 32:]])

    embs = [x0]
    sums = []
    for i in range(_L):
        acc2 = _spmm_call(xs, col2d, row2d)
        x_i, xs, s_i = _post_call(
            acc2, dinv2d, eps[i], Wmu[i], bmu[i][None], Wlv[i], blv[i][None],
            Wdec[i], bdec[i][None],
        )
        embs.append(x_i)
        sums.append(s_i)

    attWp = jnp.pad(attW, ((0, 0), (0, 128 - (_L + 1))))
    attbp = jnp.pad(attb, (0, 128 - (_L + 1)), constant_values=-1e30)[None]
    final, s3 = _final_call(
        embs[0], embs[1], embs[2], embs[3], eps[_L], Wmu[_L], bmu[_L][None],
        Wlv[_L], blv[_L][None], Wdec[_L], bdec[_L][None], attWp, attbp,
    )
    sums.append(s3)

    scores = _score_call(final, users, items + _USER)

    denom = float(_N * _D)
    losses = [s[:, 0, 0].sum() / denom - 0.5 * s[:, 0, 1].sum() / denom for s in sums]
    return scores, jnp.mean(jnp.stack(losses))


# speedup vs baseline: 10.8266x; 1.2349x over previous
import functools

import jax
import jax.numpy as jnp
from jax import lax
from jax.experimental import pallas as pl
from jax.experimental.pallas import tpu as pltpu
from jax.experimental.pallas import tpu_sc as plsc

_USER = 20000
_ITEM = 30000
_N = 50000
_E = 800000
_D = 64
_L = 3
_B = 4096

_NC = 2
_NS = 16

_CPE = 128
_ROWS = _E // _CPE
_ROWS_PAD = 6400
_TRASH = _N
_NCNT = 51200
_NACC = 50048
_NPT = _NACC // _NS

_DEG_RPT = _ROWS_PAD // (_NC * _NS)
_SP_RPT = _ROWS_PAD // _NS
_SP_STG = 80
_SP_NSTG = _SP_RPT // _SP_STG

_mesh = plsc.VectorSubcoreMesh(
    core_axis_name="c", subcore_axis_name="s", num_cores=_NC, num_subcores=_NS
)


def _deg_body(row2d, out, stage_v, ones_v, zero_v, cnt_sh):
    c = lax.axis_index("c")
    s = lax.axis_index("s")
    w = s * _NC + c

    @pl.loop(0, 200)
    def _fill_zero(i):
        zero_v[pl.ds(i * 16, 16)] = jnp.zeros((16,), jnp.float32)

    @pl.loop(0, 8)
    def _fill_one(i):
        ones_v[pl.ds(i * 16, 16)] = jnp.ones((16,), jnp.float32)

    pltpu.sync_copy(zero_v, cnt_sh.at[pl.ds(s * 3200, 3200)])
    plsc.subcore_barrier()

    pltpu.sync_copy(row2d.at[pl.ds(w * _DEG_RPT, _DEG_RPT)], stage_v)

    @pl.loop(0, _DEG_RPT)
    def _count(k):
        pltpu.sync_copy(ones_v, cnt_sh.at[stage_v.at[k]], add=True)

    plsc.subcore_barrier()
    pltpu.sync_copy(cnt_sh.at[pl.ds(s * 3200, 3200)], out.at[c, pl.ds(s * 3200, 3200)])


_deg_call = pl.kernel(
    _deg_body,
    out_type=jax.ShapeDtypeStruct((_NC, _NCNT), jnp.float32),
    mesh=_mesh,
    compiler_params=pltpu.CompilerParams(use_tc_tiling_on_sc=False, needs_layout_passes=False),
    scratch_types=[
        pltpu.VMEM((_DEG_RPT, _CPE), jnp.int32),
        pltpu.VMEM((_CPE,), jnp.float32),
        pltpu.VMEM((3200,), jnp.float32),
        pltpu.VMEM_SHARED((_NCNT,), jnp.float32),
    ],
)


def _spmm_body(xs, col2d, row2d, out, colstage, rowstage, rv0, rv1, rv2, rv3, zrow_v, acc_sh, sem, sem2):
    rows_b = (rv0, rv1, rv2, rv3)
    c = lax.axis_index("c")
    s = lax.axis_index("s")
    base = s * _SP_RPT

    @pl.loop(0, 136)
    def _fill_zero(i):
        zrow_v[i, pl.ds(0, 16)] = jnp.zeros((16,), jnp.float32)
        zrow_v[i, pl.ds(16, 16)] = jnp.zeros((16,), jnp.float32)

    @pl.loop(0, 23)
    def _zero_acc(j):
        pltpu.sync_copy(zrow_v, acc_sh.at[pl.ds(s * _NPT + j * 136, 136)])

    plsc.subcore_barrier()

    for off in range(0, _SP_RPT, 40):
        pltpu.sync_copy(col2d.at[pl.ds(base + off, 40)], colstage)
        pltpu.sync_copy(row2d.at[pl.ds(base + off, 40)], rowstage)

        @pl.loop(0, 10)
        def _grp(j):
            k0 = j * 4
            gathers = [
                pltpu.async_copy(
                    xs.at[c].at[colstage.at[k0 + b]], rows_b[b], sem
                )
                for b in range(4)
            ]
            for cp in gathers:
                cp.wait()
            scatters = [
                pltpu.async_copy(
                    rows_b[b], acc_sh.at[rowstage.at[k0 + b]], sem2, add=True
                )
                for b in range(4)
            ]
            for cp in scatters:
                cp.wait()

    plsc.subcore_barrier()
    pltpu.sync_copy(
        acc_sh.at[pl.ds(s * _NPT, _NPT)], out.at[c, pl.ds(s * _NPT, _NPT)]
    )


_spmm_call = pl.kernel(
    _spmm_body,
    out_type=jax.ShapeDtypeStruct((_NC, _NACC, 32), jnp.float32),
    mesh=_mesh,
    compiler_params=pltpu.CompilerParams(use_tc_tiling_on_sc=False, needs_layout_passes=False),
    scratch_types=[
        pltpu.VMEM((40, _CPE), jnp.int32),
        pltpu.VMEM((40, _CPE), jnp.int32),
        pltpu.VMEM((_CPE, 32), jnp.float32),
        pltpu.VMEM((_CPE, 32), jnp.float32),
        pltpu.VMEM((_CPE, 32), jnp.float32),
        pltpu.VMEM((_CPE, 32), jnp.float32),
        pltpu.VMEM((136, 32), jnp.float32),
        pltpu.VMEM_SHARED((_NACC, 32), jnp.float32),
        pltpu.SemaphoreType.DMA,
        pltpu.SemaphoreType.DMA,
    ],
)


def _score_body(final, uidx, vidx, out, iu_v, iv_v, u_v, v_v, sc_v, sem):
    c = lax.axis_index("c")
    s = lax.axis_index("s")
    w = s * _NC + c
    npair = _B // (_NC * _NS)
    base = w * npair

    pltpu.sync_copy(uidx.at[pl.ds(base, npair)], iu_v)
    pltpu.sync_copy(vidx.at[pl.ds(base, npair)], iv_v)
    cp_u = pltpu.async_copy(final.at[iu_v], u_v, sem)
    cp_v = pltpu.async_copy(final.at[iv_v], v_v, sem)
    cp_u.wait()
    cp_v.wait()

    lane = lax.iota(jnp.int32, 16)

    @pl.loop(0, npair // 16)
    def _grp(g):
        vec = jnp.zeros((16,), jnp.float32)
        for j in range(16):
            p = g * 16 + j
            d = jnp.sum(
                u_v[p, pl.ds(0, 16)] * v_v[p, pl.ds(0, 16)]
                + u_v[p, pl.ds(16, 16)] * v_v[p, pl.ds(16, 16)]
                + u_v[p, pl.ds(32, 16)] * v_v[p, pl.ds(32, 16)]
                + u_v[p, pl.ds(48, 16)] * v_v[p, pl.ds(48, 16)],
                axis=0,
            )
            vec = jnp.where(lane == j, d, vec)
        sc_v[pl.ds(g * 16, 16)] = vec

    pltpu.sync_copy(sc_v, out.at[pl.ds(base, npair)])


_score_call = pl.kernel(
    _score_body,
    out_type=jax.ShapeDtypeStruct((_B,), jnp.float32),
    mesh=_mesh,
    compiler_params=pltpu.CompilerParams(use_tc_tiling_on_sc=False, needs_layout_passes=False),
    scratch_types=[
        pltpu.VMEM((_B // (_NC * _NS),), jnp.int32),
        pltpu.VMEM((_B // (_NC * _NS),), jnp.int32),
        pltpu.VMEM((_B // (_NC * _NS), _D), jnp.float32),
        pltpu.VMEM((_B // (_NC * _NS), _D), jnp.float32),
        pltpu.VMEM((_B // (_NC * _NS),), jnp.float32),
        pltpu.SemaphoreType.DMA,
    ],
)

_R = 2000
_G = _N // _R


def _vae_sums(x, eps, wmu, bmu, wlv, blv, wdec, bdec):
    mu = x @ wmu + bmu
    lv = x @ wlv + blv
    std = jnp.exp(0.5 * lv)
    z = mu + eps * std
    recon = z @ wdec + bdec
    rsum = jnp.sum((recon - x) ** 2)
    ksum = jnp.sum(1.0 + lv - mu * mu - jnp.exp(lv))
    return rsum, ksum


def _sums_row(rsum, ksum):
    lanes = lax.broadcasted_iota(jnp.int32, (1, 1, 128), 2)
    return jnp.where(lanes == 0, rsum, jnp.where(lanes == 1, ksum, 0.0))


def _post_body(acc_ref, dinv_ref, eps_ref, wmu_ref, bmu_ref, wlv_ref, blv_ref,
               wdec_ref, bdec_ref, x_ref, xs_ref, sums_ref):
    acc = jnp.concatenate([acc_ref[0], acc_ref[1]], axis=1)
    dinv = dinv_ref[...]
    x = dinv * acc
    x_ref[...] = x
    xsn = dinv * x
    xs_ref[0] = xsn[:, :32]
    xs_ref[1] = xsn[:, 32:]
    rsum, ksum = _vae_sums(
        x, eps_ref[...], wmu_ref[...], bmu_ref[...], wlv_ref[...],
        blv_ref[...], wdec_ref[...], bdec_ref[...]
    )
    sums_ref[...] = _sums_row(rsum, ksum)


_wspec = pl.BlockSpec((_D, _D), lambda i: (0, 0))
_bspec = pl.BlockSpec((1, _D), lambda i: (0, 0))

_post_call = pl.pallas_call(
    _post_body,
    grid=(_G,),
    in_specs=[
        pl.BlockSpec((2, _R, 32), lambda i: (0, i, 0)),
        pl.BlockSpec((_R, 1), lambda i: (i, 0)),
        pl.BlockSpec((_R, _D), lambda i: (i, 0)),
        _wspec, _bspec, _wspec, _bspec, _wspec, _bspec,
    ],
    out_specs=[
        pl.BlockSpec((_R, _D), lambda i: (i, 0)),
        pl.BlockSpec((2, _R, 32), lambda i: (0, i, 0)),
        pl.BlockSpec((1, 1, 128), lambda i: (i, 0, 0)),
    ],
    out_shape=[
        jax.ShapeDtypeStruct((_N, _D), jnp.float32),
        jax.ShapeDtypeStruct((2, _N, 32), jnp.float32),
        jax.ShapeDtypeStruct((_G, 1, 128), jnp.float32),
    ],
)


def _final_body(x0_ref, x1_ref, x2_ref, x3_ref, eps_ref, wmu_ref, bmu_ref,
                wlv_ref, blv_ref, wdec_ref, bdec_ref, attw_ref, attbp_ref,
                final_ref, sums_ref):
    x0, x1, x2, x3 = x0_ref[...], x1_ref[...], x2_ref[...], x3_ref[...]
    flat = jnp.concatenate([x0, x1, x2, x3], axis=1)
    logits = flat @ attw_ref[...] + attbp_ref[...]
    m = jnp.max(logits, axis=1, keepdims=True)
    e = jnp.exp(logits - m)
    a = e / jnp.sum(e, axis=1, keepdims=True)
    final_ref[...] = (
        a[:, 0:1] * x0 + a[:, 1:2] * x1 + a[:, 2:3] * x2 + a[:, 3:4] * x3
    )
    rsum, ksum = _vae_sums(
        x3, eps_ref[...], wmu_ref[...], bmu_ref[...], wlv_ref[...],
        blv_ref[...], wdec_ref[...], bdec_ref[...]
    )
    sums_ref[...] = _sums_row(rsum, ksum)


_xspec = pl.BlockSpec((_R, _D), lambda i: (i, 0))

_final_call = pl.pallas_call(
    _final_body,
    grid=(_G,),
    in_specs=[
        _xspec, _xspec, _xspec, _xspec, _xspec,
        _wspec, _bspec, _wspec, _bspec, _wspec, _bspec,
        pl.BlockSpec(((_L + 1) * _D, 128), lambda i: (0, 0)),
        pl.BlockSpec((1, 128), lambda i: (0, 0)),
    ],
    out_specs=[
        pl.BlockSpec((_R, _D), lambda i: (i, 0)),
        pl.BlockSpec((1, 1, 128), lambda i: (i, 0, 0)),
    ],
    out_shape=[
        jax.ShapeDtypeStruct((_N, _D), jnp.float32),
        jax.ShapeDtypeStruct((_G, 1, 128), jnp.float32),
    ],
)


def kernel(users, items, edge_index, user_emb, item_emb, Wmu, bmu, Wlv, blv, Wdec, bdec, attW, attb, eps):
    row = edge_index[0]
    col = edge_index[1]
    row2d = jnp.pad(
        row.reshape(_ROWS, _CPE),
        ((0, _ROWS_PAD - _ROWS), (0, 0)),
        constant_values=_TRASH,
    )
    col2d = jnp.pad(
        col.reshape(_ROWS, _CPE), ((0, _ROWS_PAD - _ROWS), (0, 0)), constant_values=0
    )

    cnt2 = _deg_call(row2d)
    cnt = cnt2[0, :_N] + cnt2[1, :_N]
    dinv = jnp.power(cnt + 1e-10, -0.5)
    dinv2d = dinv[:, None]

    x0 = jnp.concatenate([user_emb, item_emb], axis=0)
    xsn = dinv2d * x0
    xs = jnp.stack([xsn[:, :32], xsn[:, 32:]])

    embs = [x0]
    sums = []
    for i in range(_L):
        acc2 = _spmm_call(xs, col2d, row2d)
        x_i, xs, s_i = _post_call(
            acc2, dinv2d, eps[i], Wmu[i], bmu[i][None], Wlv[i], blv[i][None],
            Wdec[i], bdec[i][None],
        )
        embs.append(x_i)
        sums.append(s_i)

    attWp = jnp.pad(attW, ((0, 0), (0, 128 - (_L + 1))))
    attbp = jnp.pad(attb, (0, 128 - (_L + 1)), constant_values=-1e30)[None]
    final, s3 = _final_call(
        embs[0], embs[1], embs[2], embs[3], eps[_L], Wmu[_L], bmu[_L][None],
        Wlv[_L], blv[_L][None], Wdec[_L], bdec[_L][None], attWp, attbp,
    )
    sums.append(s3)

    scores = _score_call(final, users, items + _USER)

    denom = float(_N * _D)
    losses = [s[:, 0, 0].sum() / denom - 0.5 * s[:, 0, 1].sum() / denom for s in sums]
    return scores, jnp.mean(jnp.stack(losses))


# speedup vs baseline: 11.2353x; 1.0377x over previous
import functools

import jax
import jax.numpy as jnp
from jax import lax
from jax.experimental import pallas as pl
from jax.experimental.pallas import tpu as pltpu
from jax.experimental.pallas import tpu_sc as plsc

_USER = 20000
_ITEM = 30000
_N = 50000
_E = 800000
_D = 64
_L = 3
_B = 4096

_NC = 2
_NS = 16

_CPE = 128
_ROWS = _E // _CPE
_ROWS_PAD = 6400
_TRASH = _N
_NCNT = 51200
_NACC = 50048
_NPT = _NACC // _NS

_DEG_RPT = _ROWS_PAD // (_NC * _NS)
_SP_RPT = _ROWS_PAD // _NS
_SP_STG = 80
_SP_NSTG = _SP_RPT // _SP_STG

_mesh = plsc.VectorSubcoreMesh(
    core_axis_name="c", subcore_axis_name="s", num_cores=_NC, num_subcores=_NS
)


def _deg_body(row2d, out, stage_v, ones_v, zero_v, cnt_sh):
    c = lax.axis_index("c")
    s = lax.axis_index("s")
    w = s * _NC + c

    @pl.loop(0, 200)
    def _fill_zero(i):
        zero_v[pl.ds(i * 16, 16)] = jnp.zeros((16,), jnp.float32)

    @pl.loop(0, 8)
    def _fill_one(i):
        ones_v[pl.ds(i * 16, 16)] = jnp.ones((16,), jnp.float32)

    pltpu.sync_copy(zero_v, cnt_sh.at[pl.ds(s * 3200, 3200)])
    plsc.subcore_barrier()

    pltpu.sync_copy(row2d.at[pl.ds(w * _DEG_RPT, _DEG_RPT)], stage_v)

    @pl.loop(0, _DEG_RPT)
    def _count(k):
        pltpu.sync_copy(ones_v, cnt_sh.at[stage_v.at[k]], add=True)

    plsc.subcore_barrier()
    pltpu.sync_copy(cnt_sh.at[pl.ds(s * 3200, 3200)], out.at[c, pl.ds(s * 3200, 3200)])


_deg_call = pl.kernel(
    _deg_body,
    out_type=jax.ShapeDtypeStruct((_NC, _NCNT), jnp.float32),
    mesh=_mesh,
    compiler_params=pltpu.CompilerParams(use_tc_tiling_on_sc=False, needs_layout_passes=False),
    scratch_types=[
        pltpu.VMEM((_DEG_RPT, _CPE), jnp.int32),
        pltpu.VMEM((_CPE,), jnp.float32),
        pltpu.VMEM((3200,), jnp.float32),
        pltpu.VMEM_SHARED((_NCNT,), jnp.float32),
    ],
)


def _spmm_body(xs, col2d, row2d, out, colstage, rowstage, rv0, rv1, rv2, rv3, zrow_v, acc_sh, sem, sem2):
    rows_b = (rv0, rv1, rv2, rv3)
    c = lax.axis_index("c")
    s = lax.axis_index("s")
    base = s * _SP_RPT

    @pl.loop(0, 136)
    def _fill_zero(i):
        zrow_v[i, pl.ds(0, 16)] = jnp.zeros((16,), jnp.float32)
        zrow_v[i, pl.ds(16, 16)] = jnp.zeros((16,), jnp.float32)

    @pl.loop(0, 23)
    def _zero_acc(j):
        pltpu.sync_copy(zrow_v, acc_sh.at[pl.ds(s * _NPT + j * 136, 136)])

    plsc.subcore_barrier()

    for off in range(0, _SP_RPT, 40):
        pltpu.sync_copy(col2d.at[pl.ds(base + off, 40)], colstage)
        pltpu.sync_copy(row2d.at[pl.ds(base + off, 40)], rowstage)

        @pl.loop(0, 10)
        def _grp(j):
            k0 = j * 4
            gathers = [
                pltpu.async_copy(
                    xs.at[c].at[colstage.at[k0 + b]], rows_b[b], sem.at[b]
                )
                for b in range(4)
            ]
            scatters = []
            for b in range(4):
                gathers[b].wait()
                scatters.append(
                    pltpu.async_copy(
                        rows_b[b], acc_sh.at[rowstage.at[k0 + b]], sem2,
                        add=True,
                    )
                )
            for cp in scatters:
                cp.wait()

    plsc.subcore_barrier()
    pltpu.sync_copy(
        acc_sh.at[pl.ds(s * _NPT, _NPT)], out.at[c, pl.ds(s * _NPT, _NPT)]
    )


_spmm_call = pl.kernel(
    _spmm_body,
    out_type=jax.ShapeDtypeStruct((_NC, _NACC, 32), jnp.float32),
    mesh=_mesh,
    compiler_params=pltpu.CompilerParams(use_tc_tiling_on_sc=False, needs_layout_passes=False),
    scratch_types=[
        pltpu.VMEM((40, _CPE), jnp.int32),
        pltpu.VMEM((40, _CPE), jnp.int32),
        pltpu.VMEM((_CPE, 32), jnp.float32),
        pltpu.VMEM((_CPE, 32), jnp.float32),
        pltpu.VMEM((_CPE, 32), jnp.float32),
        pltpu.VMEM((_CPE, 32), jnp.float32),
        pltpu.VMEM((136, 32), jnp.float32),
        pltpu.VMEM_SHARED((_NACC, 32), jnp.float32),
        pltpu.SemaphoreType.DMA((4,)),
        pltpu.SemaphoreType.DMA,
    ],
)


def _score_body(final, uidx, vidx, out, iu_v, iv_v, u_v, v_v, sc_v, sem):
    c = lax.axis_index("c")
    s = lax.axis_index("s")
    w = s * _NC + c
    npair = _B // (_NC * _NS)
    base = w * npair

    pltpu.sync_copy(uidx.at[pl.ds(base, npair)], iu_v)
    pltpu.sync_copy(vidx.at[pl.ds(base, npair)], iv_v)
    cp_u = pltpu.async_copy(final.at[iu_v], u_v, sem)
    cp_v = pltpu.async_copy(final.at[iv_v], v_v, sem)
    cp_u.wait()
    cp_v.wait()

    lane = lax.iota(jnp.int32, 16)

    @pl.loop(0, npair // 16)
    def _grp(g):
        vec = jnp.zeros((16,), jnp.float32)
        for j in range(16):
            p = g * 16 + j
            d = jnp.sum(
                u_v[p, pl.ds(0, 16)] * v_v[p, pl.ds(0, 16)]
                + u_v[p, pl.ds(16, 16)] * v_v[p, pl.ds(16, 16)]
                + u_v[p, pl.ds(32, 16)] * v_v[p, pl.ds(32, 16)]
                + u_v[p, pl.ds(48, 16)] * v_v[p, pl.ds(48, 16)],
                axis=0,
            )
            vec = jnp.where(lane == j, d, vec)
        sc_v[pl.ds(g * 16, 16)] = vec

    pltpu.sync_copy(sc_v, out.at[pl.ds(base, npair)])


_score_call = pl.kernel(
    _score_body,
    out_type=jax.ShapeDtypeStruct((_B,), jnp.float32),
    mesh=_mesh,
    compiler_params=pltpu.CompilerParams(use_tc_tiling_on_sc=False, needs_layout_passes=False),
    scratch_types=[
        pltpu.VMEM((_B // (_NC * _NS),), jnp.int32),
        pltpu.VMEM((_B // (_NC * _NS),), jnp.int32),
        pltpu.VMEM((_B // (_NC * _NS), _D), jnp.float32),
        pltpu.VMEM((_B // (_NC * _NS), _D), jnp.float32),
        pltpu.VMEM((_B // (_NC * _NS),), jnp.float32),
        pltpu.SemaphoreType.DMA,
    ],
)

_R = 5000
_G = _N // _R


def _vae_sums(x, eps, wmu, bmu, wlv, blv, wdec, bdec):
    mu = x @ wmu + bmu
    lv = x @ wlv + blv
    std = jnp.exp(0.5 * lv)
    z = mu + eps * std
    recon = z @ wdec + bdec
    rsum = jnp.sum((recon - x) ** 2)
    ksum = jnp.sum(1.0 + lv - mu * mu - jnp.exp(lv))
    return rsum, ksum


def _sums_row(rsum, ksum):
    lanes = lax.broadcasted_iota(jnp.int32, (1, 1, 128), 2)
    return jnp.where(lanes == 0, rsum, jnp.where(lanes == 1, ksum, 0.0))


def _post_body(acc_ref, dinv_ref, eps_ref, wmu_ref, bmu_ref, wlv_ref, blv_ref,
               wdec_ref, bdec_ref, x_ref, xs_ref, sums_ref):
    acc = jnp.concatenate([acc_ref[0], acc_ref[1]], axis=1)
    dinv = dinv_ref[...]
    x = dinv * acc
    x_ref[...] = x
    xsn = dinv * x
    xs_ref[0] = xsn[:, :32]
    xs_ref[1] = xsn[:, 32:]
    rsum, ksum = _vae_sums(
        x, eps_ref[...], wmu_ref[...], bmu_ref[...], wlv_ref[...],
        blv_ref[...], wdec_ref[...], bdec_ref[...]
    )
    sums_ref[...] = _sums_row(rsum, ksum)


_wspec = pl.BlockSpec((_D, _D), lambda i: (0, 0))
_bspec = pl.BlockSpec((1, _D), lambda i: (0, 0))

_post_call = pl.pallas_call(
    _post_body,
    grid=(_G,),
    in_specs=[
        pl.BlockSpec((2, _R, 32), lambda i: (0, i, 0)),
        pl.BlockSpec((_R, 1), lambda i: (i, 0)),
        pl.BlockSpec((_R, _D), lambda i: (i, 0)),
        _wspec, _bspec, _wspec, _bspec, _wspec, _bspec,
    ],
    out_specs=[
        pl.BlockSpec((_R, _D), lambda i: (i, 0)),
        pl.BlockSpec((2, _R, 32), lambda i: (0, i, 0)),
        pl.BlockSpec((1, 1, 128), lambda i: (i, 0, 0)),
    ],
    out_shape=[
        jax.ShapeDtypeStruct((_N, _D), jnp.float32),
        jax.ShapeDtypeStruct((2, _N, 32), jnp.float32),
        jax.ShapeDtypeStruct((_G, 1, 128), jnp.float32),
    ],
)


def _final_body(x0_ref, x1_ref, x2_ref, x3_ref, eps_ref, wmu_ref, bmu_ref,
                wlv_ref, blv_ref, wdec_ref, bdec_ref, attw_ref, attbp_ref,
                final_ref, sums_ref):
    x0, x1, x2, x3 = x0_ref[...], x1_ref[...], x2_ref[...], x3_ref[...]
    flat = jnp.concatenate([x0, x1, x2, x3], axis=1)
    logits = flat @ attw_ref[...] + attbp_ref[...]
    m = jnp.max(logits, axis=1, keepdims=True)
    e = jnp.exp(logits - m)
    a = e / jnp.sum(e, axis=1, keepdims=True)
    final_ref[...] = (
        a[:, 0:1] * x0 + a[:, 1:2] * x1 + a[:, 2:3] * x2 + a[:, 3:4] * x3
    )
    rsum, ksum = _vae_sums(
        x3, eps_ref[...], wmu_ref[...], bmu_ref[...], wlv_ref[...],
        blv_ref[...], wdec_ref[...], bdec_ref[...]
    )
    sums_ref[...] = _sums_row(rsum, ksum)


_xspec = pl.BlockSpec((_R, _D), lambda i: (i, 0))

_final_call = pl.pallas_call(
    _final_body,
    grid=(_G,),
    in_specs=[
        _xspec, _xspec, _xspec, _xspec, _xspec,
        _wspec, _bspec, _wspec, _bspec, _wspec, _bspec,
        pl.BlockSpec(((_L + 1) * _D, 128), lambda i: (0, 0)),
        pl.BlockSpec((1, 128), lambda i: (0, 0)),
    ],
    out_specs=[
        pl.BlockSpec((_R, _D), lambda i: (i, 0)),
        pl.BlockSpec((1, 1, 128), lambda i: (i, 0, 0)),
    ],
    out_shape=[
        jax.ShapeDtypeStruct((_N, _D), jnp.float32),
        jax.ShapeDtypeStruct((_G, 1, 128), jnp.float32),
    ],
)


def kernel(users, items, edge_index, user_emb, item_emb, Wmu, bmu, Wlv, blv, Wdec, bdec, attW, attb, eps):
    row = edge_index[0]
    col = edge_index[1]
    row2d = jnp.pad(
        row.reshape(_ROWS, _CPE),
        ((0, _ROWS_PAD - _ROWS), (0, 0)),
        constant_values=_TRASH,
    )
    col2d = jnp.pad(
        col.reshape(_ROWS, _CPE), ((0, _ROWS_PAD - _ROWS), (0, 0)), constant_values=0
    )

    cnt2 = _deg_call(row2d)
    cnt = cnt2[0, :_N] + cnt2[1, :_N]
    dinv = jnp.power(cnt + 1e-10, -0.5)
    dinv2d = dinv[:, None]

    x0 = jnp.concatenate([user_emb, item_emb], axis=0)
    xsn = dinv2d * x0
    xs = jnp.stack([xsn[:, :32], xsn[:, 32:]])

    embs = [x0]
    sums = []
    for i in range(_L):
        acc2 = _spmm_call(xs, col2d, row2d)
        x_i, xs, s_i = _post_call(
            acc2, dinv2d, eps[i], Wmu[i], bmu[i][None], Wlv[i], blv[i][None],
            Wdec[i], bdec[i][None],
        )
        embs.append(x_i)
        sums.append(s_i)

    attWp = jnp.pad(attW, ((0, 0), (0, 128 - (_L + 1))))
    attbp = jnp.pad(attb, (0, 128 - (_L + 1)), constant_values=-1e30)[None]
    final, s3 = _final_call(
        embs[0], embs[1], embs[2], embs[3], eps[_L], Wmu[_L], bmu[_L][None],
        Wlv[_L], blv[_L][None], Wdec[_L], bdec[_L][None], attWp, attbp,
    )
    sums.append(s3)

    scores = _score_call(final, users, items + _USER)

    denom = float(_N * _D)
    losses = [s[:, 0, 0].sum() / denom - 0.5 * s[:, 0, 1].sum() / denom for s in sums]
    return scores, jnp.mean(jnp.stack(losses))
